# baseline (speedup 1.0000x reference)
import functools

import jax
import jax.numpy as jnp
from jax import lax
from jax.experimental import pallas as pl
from jax.experimental.pallas import tpu as pltpu
from jax.experimental.pallas import tpu_sc as plsc

N, C, H, W = 5000, 4, 56, 56
HW = H * W
NPAD = 5120
NT = 8
NMAIN = N - NT
L = 16
NW = 32
HB = 8
NOFF = (0, 640, 1280, 1920, 2560, 3200, 3840, 4480)
NBQ = (640, 640, 640, 640, 640, 640, 640, 512)
NB = NBQ[0]
NQ = len(NBQ)
NBATCH = HW // (NW * HB)
XTRA = HW // HB - NW * NBATCH
HMAX = (NBATCH + 1) * HB

_mesh = plsc.VectorSubcoreMesh(core_axis_name="c", subcore_axis_name="s")


@functools.partial(
    pl.kernel,
    mesh=_mesh,
    compiler_params=pltpu.CompilerParams(needs_layout_passes=False),
    out_type=jax.ShapeDtypeStruct((HW, NPAD), jnp.float32),
    scratch_types=[
        pltpu.VMEM((NPAD,), jnp.int32),
        pltpu.VMEM((C, HB, NB), jnp.float32),
        pltpu.VMEM((C, HB, NB), jnp.float32),
        pltpu.VMEM((HB, NB), jnp.float32),
        pltpu.VMEM((HB, NB), jnp.float32),
        pltpu.VMEM((HB, NB), jnp.float32),
        pltpu.VMEM((HB, NB), jnp.float32),
        pltpu.VMEM((C, HMAX, NT), jnp.float32),
        pltpu.VMEM((HB, 128), jnp.float32),
        pltpu.VMEM((HB, 128), jnp.float32),
        pltpu.SemaphoreType.DMA,
        pltpu.SemaphoreType.DMA,
        pltpu.SemaphoreType.DMA,
        pltpu.SemaphoreType.DMA,
        pltpu.SemaphoreType.DMA,
        pltpu.SemaphoreType.DMA,
        pltpu.SemaphoreType.DMA,
        pltpu.SemaphoreType.DMA,
    ],
)
def _select_kernel(xt_hbm, xtail_hbm, labels_hbm, out_hbm, lbl_v,
                   ibuf0, ibuf1, obuf0, obuf1, obuf2, obuf3,
                   ibuf_t, obuf_t0, obuf_t1,
                   g0, g1, w0, w1, w2, w3, gt, wt):
    wid = lax.axis_index("s") * 2 + lax.axis_index("c")
    has_extra = wid < XTRA
    hw_start = wid * (NBATCH * HB) + HB * jnp.minimum(wid, XTRA)
    nb = NBATCH + jnp.where(has_extra, 1, 0)
    pltpu.sync_copy(labels_hbm, lbl_v)
    iota = lax.iota(jnp.int32, L)
    tail_mask = iota < NT
    tail_idx = jnp.minimum(iota, NT - 1)

    ibufs = (ibuf0, ibuf1)
    obufs = (obuf0, obuf1, obuf2, obuf3)
    gsems = (g0, g1)
    wsems = (w0, w1, w2, w3)

    def compute(ibuf, obuf, n0, ngroups):
        def kbody(k, _):
            lbl16 = lbl_v[pl.ds(n0 + k * L, L)]
            n_idx = k * L + iota
            vs = [
                plsc.load_gather(
                    ibuf, [lbl16, jnp.full((L,), h, jnp.int32), n_idx])
                for h in range(HB)
            ]
            for h in range(HB):
                obuf[h, pl.ds(k * L, L)] = vs[h]
            return 0
        lax.fori_loop(0, ngroups, kbody, 0, unroll=4)

    def fire_in(hw0, q):
        return pltpu.async_copy(
            xt_hbm.at[:, pl.ds(hw0, HB), pl.ds(NOFF[q], NBQ[q])],
            ibufs[q % 2].at[:, :, pl.ds(0, NBQ[q])], gsems[q % 2])

    def drain_in0():
        pltpu.make_async_copy(
            xt_hbm.at[:, pl.ds(0, HB), pl.ds(0, NB)],
            ibufs[0], gsems[0]).wait()

    def drain_out(s, size):
        pltpu.make_async_copy(
            out_hbm.at[pl.ds(0, HB), pl.ds(0, size)],
            obufs[s].at[:, pl.ds(0, size)], wsems[s]).wait()

    def fire_out(hw0, q):
        pltpu.async_copy(
            obufs[q % 4].at[:, pl.ds(0, NBQ[q])],
            out_hbm.at[pl.ds(hw0, HB), pl.ds(NOFF[q], NBQ[q])],
            wsems[q % 4])

    def run_block(hw0, q, reclaim_size, reclaim_pred=None):
        s = q % 4
        if reclaim_pred is None:
            drain_out(s, reclaim_size)
        else:
            @pl.when(reclaim_pred)
            def _():
                drain_out(s, reclaim_size)
        compute(ibufs[q % 2], obufs[s], NOFF[q], NBQ[q] // L)
        fire_out(hw0, q)

    fire_in(hw_start, 0)

    def batch_body(b, _):
        hw0 = hw_start + b * HB
        pending = fire_in(hw0, 1)
        drain_in0()
        run_block(hw0, 0, NBQ[NQ - 4], b > 0)
        for q in range(1, NQ):
            nxt = fire_in(hw0, q + 1) if q + 1 < NQ else None
            if q + 1 == NQ:
                @pl.when(b + 1 < nb)
                def _():
                    fire_in(hw0 + HB, 0)
            pending.wait()
            if q < 4:
                run_block(hw0, q, NBQ[NQ - 4 + q], b > 0)
            else:
                run_block(hw0, q, NBQ[q - 4])
            pending = nxt
        return 0

    lax.fori_loop(0, nb, batch_body, 0)
    for s in range(4):
        drain_out(s, NBQ[NQ - 4 + s])

    obuf_ts = (obuf_t0, obuf_t1)
    wsem_ts = (wt, gt)
    lbl_tail = lbl_v[pl.ds(NMAIN, L)]

    def drain_tail(s):
        pltpu.make_async_copy(
            out_hbm.at[pl.ds(0, HB), pl.ds(0, 128)], obuf_ts[s],
            wsem_ts[s]).wait()

    def tail(nrows):
        pltpu.async_copy(
            xtail_hbm.at[:, pl.ds(hw_start, nrows)],
            ibuf_t.at[:, pl.ds(0, nrows)], gt).wait()
        nbt = nrows // HB
        for i in range(nbt):
            if i >= 2:
                drain_tail(i % 2)
            ob = obuf_ts[i % 2]
            for j in range(HB):
                h_idx = jnp.full((L,), i * HB + j, jnp.int32)
                v = plsc.load_gather(ibuf_t, [lbl_tail, h_idx, tail_idx],
                                     mask=tail_mask)
                plsc.store_scatter(ob, [jnp.full((L,), j, jnp.int32),
                                        tail_idx], v, mask=tail_mask)
            pltpu.async_copy(
                ob, out_hbm.at[pl.ds(hw_start + i * HB, HB),
                               pl.ds(NMAIN, 128)], wsem_ts[i % 2])
        for s in range(2):
            drain_tail((nbt + s) % 2)

    @pl.when(has_extra)
    def _():
        tail(HMAX)

    @pl.when(jnp.logical_not(has_extra))
    def _():
        tail(NBATCH * HB)


def kernel(x, labels):
    xt = jnp.transpose(x, (1, 2, 3, 0)).reshape(C, HW, N)
    xtail = jnp.transpose(x[NMAIN:], (1, 2, 3, 0)).reshape(C, HW, NT)
    lbl = jnp.pad(labels.astype(jnp.int32), (0, NPAD - N))
    out2 = _select_kernel(xt, xtail, lbl)
    return jnp.transpose(out2[:, :N].reshape(H, W, N)[None], (3, 0, 1, 2))


# speedup vs baseline: 1.0411x; 1.0411x over previous
import functools

import jax
import jax.numpy as jnp
from jax import lax
from jax.experimental import pallas as pl
from jax.experimental.pallas import tpu as pltpu
from jax.experimental.pallas import tpu_sc as plsc

N, C, H, W = 5000, 4, 56, 56
HW = H * W
NPAD = 5120
NT = 8
NMAIN = N - NT
L = 16
NW = 32
HB = 8
NBQ = (384,) * 7 + (256,) * 9
NOFF = tuple(sum(NBQ[:i]) for i in range(len(NBQ)))
NB = NBQ[0]
NQ = len(NBQ)
NBATCH = HW // (NW * HB)
XTRA = HW // HB - NW * NBATCH
HMAX = (NBATCH + 1) * HB

_mesh = plsc.VectorSubcoreMesh(core_axis_name="c", subcore_axis_name="s")


@functools.partial(
    pl.kernel,
    mesh=_mesh,
    compiler_params=pltpu.CompilerParams(needs_layout_passes=False),
    out_type=jax.ShapeDtypeStruct((HW, NPAD), jnp.float32),
    scratch_types=[
        pltpu.VMEM((NPAD,), jnp.int32),
        pltpu.VMEM((C, HB, NB), jnp.float32),
        pltpu.VMEM((C, HB, NB), jnp.float32),
        pltpu.VMEM((C, HB, NB), jnp.float32),
        pltpu.VMEM((C, HB, NB), jnp.float32),
        pltpu.VMEM((HB, NB), jnp.float32),
        pltpu.VMEM((HB, NB), jnp.float32),
        pltpu.VMEM((HB, NB), jnp.float32),
        pltpu.VMEM((HB, NB), jnp.float32),
        pltpu.VMEM((C, HMAX, NT), jnp.float32),
        pltpu.VMEM((HB, 128), jnp.float32),
        pltpu.VMEM((HB, 128), jnp.float32),
        pltpu.SemaphoreType.DMA,
        pltpu.SemaphoreType.DMA,
        pltpu.SemaphoreType.DMA,
        pltpu.SemaphoreType.DMA,
        pltpu.SemaphoreType.DMA,
        pltpu.SemaphoreType.DMA,
        pltpu.SemaphoreType.DMA,
        pltpu.SemaphoreType.DMA,
        pltpu.SemaphoreType.DMA,
        pltpu.SemaphoreType.DMA,
    ],
)
def _select_kernel(xt_hbm, xtail_hbm, labels_hbm, out_hbm, lbl_v,
                   ibuf0, ibuf1, ibuf2, ibuf3, obuf0, obuf1, obuf2, obuf3,
                   ibuf_t, obuf_t0, obuf_t1,
                   g0, g1, g2, g3, w0, w1, w2, w3, gt, wt):
    wid = lax.axis_index("s") * 2 + lax.axis_index("c")
    has_extra = wid < XTRA
    hw_start = wid * (NBATCH * HB) + HB * jnp.minimum(wid, XTRA)
    nb = NBATCH + jnp.where(has_extra, 1, 0)
    pltpu.sync_copy(labels_hbm, lbl_v)
    iota = lax.iota(jnp.int32, L)
    tail_mask = iota < NT
    tail_idx = jnp.minimum(iota, NT - 1)

    ibufs = (ibuf0, ibuf1, ibuf2, ibuf3)
    obufs = (obuf0, obuf1, obuf2, obuf3)
    gsems = (g0, g1, g2, g3)
    wsems = (w0, w1, w2, w3)

    def compute(ibuf, obuf, n0, ngroups):
        def kbody(k, _):
            lbl16 = lbl_v[pl.ds(n0 + k * L, L)]
            n_idx = k * L + iota
            vs = [
                plsc.load_gather(
                    ibuf, [lbl16, jnp.full((L,), h, jnp.int32), n_idx])
                for h in range(HB)
            ]
            for h in range(HB):
                obuf[h, pl.ds(k * L, L)] = vs[h]
            return 0
        lax.fori_loop(0, ngroups, kbody, 0, unroll=4)

    def fire_in(hw0, q):
        return pltpu.async_copy(
            xt_hbm.at[:, pl.ds(hw0, HB), pl.ds(NOFF[q], NBQ[q])],
            ibufs[q % 4].at[:, :, pl.ds(0, NBQ[q])], gsems[q % 4])

    def drain_in(q):
        pltpu.make_async_copy(
            xt_hbm.at[:, pl.ds(0, HB), pl.ds(0, NBQ[q])],
            ibufs[q % 4].at[:, :, pl.ds(0, NBQ[q])], gsems[q % 4]).wait()

    def drain_out(s, size):
        pltpu.make_async_copy(
            out_hbm.at[pl.ds(0, HB), pl.ds(0, size)],
            obufs[s].at[:, pl.ds(0, size)], wsems[s]).wait()

    def fire_out(hw0, q):
        pltpu.async_copy(
            obufs[q % 4].at[:, pl.ds(0, NBQ[q])],
            out_hbm.at[pl.ds(hw0, HB), pl.ds(NOFF[q], NBQ[q])],
            wsems[q % 4])

    def run_block(hw0, q, reclaim_size, reclaim_pred=None):
        s = q % 4
        if reclaim_pred is None:
            drain_out(s, reclaim_size)
        else:
            @pl.when(reclaim_pred)
            def _():
                drain_out(s, reclaim_size)
        compute(ibufs[s], obufs[s], NOFF[q], NBQ[q] // L)
        fire_out(hw0, q)

    fire_in(hw_start, 0)
    fire_in(hw_start, 1)

    def batch_body(b, _):
        hw0 = hw_start + b * HB
        handles = {}
        for q in range(NQ):
            if q + 2 < NQ:
                handles[q + 2] = fire_in(hw0, q + 2)
            else:
                @pl.when(b + 1 < nb)
                def _(q=q):
                    fire_in(hw0 + HB, q + 2 - NQ)
            if q in handles:
                handles.pop(q).wait()
            else:
                drain_in(q)
            if q < 4:
                run_block(hw0, q, NBQ[NQ - 4 + q], b > 0)
            else:
                run_block(hw0, q, NBQ[q - 4])
        return 0

    lax.fori_loop(0, nb, batch_body, 0)
    for s in range(4):
        drain_out(s, NBQ[NQ - 4 + s])

    obuf_ts = (obuf_t0, obuf_t1)
    wsem_ts = (wt, gt)
    lbl_tail = lbl_v[pl.ds(NMAIN, L)]

    def drain_tail(s):
        pltpu.make_async_copy(
            out_hbm.at[pl.ds(0, HB), pl.ds(0, 128)], obuf_ts[s],
            wsem_ts[s]).wait()

    def tail(nrows):
        pltpu.async_copy(
            xtail_hbm.at[:, pl.ds(hw_start, nrows)],
            ibuf_t.at[:, pl.ds(0, nrows)], gt).wait()
        nbt = nrows // HB
        for i in range(nbt):
            if i >= 2:
                drain_tail(i % 2)
            ob = obuf_ts[i % 2]
            for j in range(HB):
                h_idx = jnp.full((L,), i * HB + j, jnp.int32)
                v = plsc.load_gather(ibuf_t, [lbl_tail, h_idx, tail_idx],
                                     mask=tail_mask)
                plsc.store_scatter(ob, [jnp.full((L,), j, jnp.int32),
                                        tail_idx], v, mask=tail_mask)
            pltpu.async_copy(
                ob, out_hbm.at[pl.ds(hw_start + i * HB, HB),
                               pl.ds(NMAIN, 128)], wsem_ts[i % 2])
        for s in range(2):
            drain_tail((nbt + s) % 2)

    @pl.when(has_extra)
    def _():
        tail(HMAX)

    @pl.when(jnp.logical_not(has_extra))
    def _():
        tail(NBATCH * HB)


def kernel(x, labels):
    xt = jnp.transpose(x, (1, 2, 3, 0)).reshape(C, HW, N)
    xtail = jnp.transpose(x[NMAIN:], (1, 2, 3, 0)).reshape(C, HW, NT)
    lbl = jnp.pad(labels.astype(jnp.int32), (0, NPAD - N))
    out2 = _select_kernel(xt, xtail, lbl)
    return jnp.transpose(out2[:, :N].reshape(H, W, N)[None], (3, 0, 1, 2))
